# baseline (speedup 1.0000x reference)
import jax
import jax.numpy as jnp
from jax import lax
from jax.experimental import pallas as pl
from jax.experimental.pallas import tpu as pltpu
from jax.experimental.pallas import tpu_sc as plsc

B = 16
L = 4096
D = 1024
NC = 2
NS = 16
DH = D // NC
CHUNK = 32
NLANE = 16
NCG = DH // NLANE

TROWS = 512
KMAX = L // TROWS


def _sc_body(features, s_hbm, out, s_v, buf, part, red, shared, final, sems):
    c = lax.axis_index("c")
    s = lax.axis_index("s")
    col0 = c * DH

    pltpu.sync_copy(s_hbm, s_v)

    iota = lax.iota(jnp.int32, NLANE)
    s_vec = s_v[...]
    zero = jnp.zeros((NLANE,), jnp.float32)

    def zero_row(i, _):
        def zero_cg(g, _):
            part[i, pl.ds(g * NLANE, NLANE)] = zero
            return 0
        return lax.fori_loop(0, NCG, zero_cg, 0)

    lax.fori_loop(0, B, zero_row, 0)

    nblk_v = s_vec // 8
    r0_v = 8 * ((s * nblk_v) // NS)
    rb1_v = 8 * (((s + 1) * nblk_v) // NS)
    nch_v = (rb1_v - r0_v + CHUNK - 1) // CHUNK
    total = jnp.sum(nch_v)

    def geti(v, i):
        return jnp.sum(jnp.where(iota == i, v, 0))

    def advance(i, k):
        def cond(st):
            i_, k_ = st
            return (i_ < B) & (k_ >= geti(nch_v, i_))

        def step(st):
            i_, _ = st
            return (i_ + 1, 0)

        return lax.while_loop(cond, step, (i, k))

    def chunk_start(i, k):
        r0 = geti(r0_v, i)
        rb1 = geti(rb1_v, i)
        islast = k == geti(nch_v, i) - 1
        start = jnp.where(
            islast, jnp.maximum(rb1 - CHUNK, 0), r0 + k * CHUNK
        )
        return pl.multiple_of(start, 8)

    def issue(i, k, par):
        return pltpu.async_copy(
            features.at[i, pl.ds(chunk_start(i, k), CHUNK), pl.ds(col0, DH)],
            buf.at[par],
            sems.at[par],
        )

    c0 = advance(0, 0)
    c1 = advance(c0[0], c0[1] + 1)
    c2 = advance(c1[0], c1[1] + 1)
    c3 = advance(c2[0], c2[1] + 1)
    for depth, cc in enumerate((c0, c1, c2)):
        @pl.when(total > depth)
        def _(cc=cc, depth=depth):
            issue(cc[0], cc[1], depth)

    def chunk_step(m, st):
        i, k, n1, n2, n3 = st
        par = m & 3

        @pl.when(m + 3 < total)
        def _():
            issue(n3[0], n3[1], (m + 3) & 3)

        pltpu.make_async_copy(
            features.at[i, pl.ds(chunk_start(i, k), CHUNK), pl.ds(col0, DH)],
            buf.at[par],
            sems.at[par],
        ).wait()

        islast = k == geti(nch_v, i) - 1

        @pl.when(jnp.logical_not(islast))
        def _():
            def per_cg(g, _):
                base = g * NLANE
                acc = part[i, pl.ds(base, NLANE)]
                for j in range(CHUNK):
                    acc = acc + buf[par, j, pl.ds(base, NLANE)]
                part[i, pl.ds(base, NLANE)] = acc
                return 0

            lax.fori_loop(0, NCG, per_cg, 0)

        @pl.when(islast)
        def _():
            start = chunk_start(i, k)
            lo = geti(r0_v, i) + k * CHUNK
            hi = geti(rb1_v, i)
            scales = [
                jnp.where((start + j >= lo) & (start + j < hi), 1.0, 0.0)
                for j in range(CHUNK)
            ]

            def per_cg(g, _):
                base = g * NLANE
                acc = part[i, pl.ds(base, NLANE)]
                for j in range(CHUNK):
                    acc = acc + buf[par, j, pl.ds(base, NLANE)] * scales[j]
                part[i, pl.ds(base, NLANE)] = acc
                return 0

            lax.fori_loop(0, NCG, per_cg, 0)

        return (n1[0], n1[1], n2, n3, advance(n3[0], n3[1] + 1))

    lax.fori_loop(0, total, chunk_step, (c0[0], c0[1], c1, c2, c3))

    pltpu.sync_copy(part, shared.at[s])
    plsc.subcore_barrier()

    pltpu.sync_copy(shared.at[:, s], part)

    def red_cg(g, _):
        base = g * NLANE
        acc = part[0, pl.ds(base, NLANE)]
        for w in range(1, NS):
            acc = acc + part[w, pl.ds(base, NLANE)]
        red[pl.ds(base, NLANE)] = acc
        return 0

    lax.fori_loop(0, NCG, red_cg, 0)

    pltpu.sync_copy(red, final.at[s])
    plsc.subcore_barrier()

    @pl.when(s == 0)
    def _():
        pltpu.sync_copy(final, out.at[:, pl.ds(col0, DH)])


def _tc_body(lob, hib, effs, feats_ref, out_ref):
    k = pl.program_id(1)
    i = pl.program_id(0)

    @pl.when(k == 0)
    def _():
        out_ref[...] = jnp.zeros(out_ref.shape, out_ref.dtype)

    lo = lob[i]
    hi = hib[i]

    @pl.when(lo + k < hi)
    def _():
        bidx = lo + k
        rows = bidx * TROWS + lax.broadcasted_iota(jnp.int32, (1, TROWS), 1)
        mask = (rows < effs[i]).astype(jnp.float32)
        out_ref[...] += jnp.dot(
            mask, feats_ref[0], preferred_element_type=jnp.float32
        )[None]


def _combine_body(a_ref, b_ref, inv_ref, o_ref):
    o_ref[...] = (a_ref[...] + b_ref[...]) * inv_ref[...]


@jax.jit
def kernel(features, lengths):
    eff = jnp.minimum(jnp.where(lengths <= 0, L, lengths), L).astype(jnp.int32)
    inv = 1.0 / eff.astype(jnp.float32)
    sblk = jnp.minimum((7 * eff + 5 * TROWS) // (10 * TROWS), eff // TROWS)
    s_rows = (TROWS // 8 * sblk) * 8

    mesh = plsc.VectorSubcoreMesh(core_axis_name="c", subcore_axis_name="s")
    sc_run = pl.kernel(
        _sc_body,
        out_type=jax.ShapeDtypeStruct((B, D), jnp.float32),
        mesh=mesh,
        scratch_types=[
            pltpu.VMEM((B,), jnp.int32),
            pltpu.VMEM((4, CHUNK, DH), jnp.float32),
            pltpu.VMEM((B, DH), jnp.float32),
            pltpu.VMEM((DH,), jnp.float32),
            pltpu.VMEM_SHARED((NS, B, DH), jnp.float32),
            pltpu.VMEM_SHARED((B, DH), jnp.float32),
            pltpu.SemaphoreType.DMA((4,)),
        ],
        compiler_params=pltpu.CompilerParams(needs_layout_passes=False),
    )
    sc_sums = sc_run(features, s_rows)

    hib = (eff + TROWS - 1) // TROWS
    tc_sums = pl.pallas_call(
        _tc_body,
        grid_spec=pltpu.PrefetchScalarGridSpec(
            num_scalar_prefetch=3,
            grid=(B, KMAX),
            in_specs=[
                pl.BlockSpec(
                    (1, TROWS, D),
                    lambda i, k, lob, hib, effs: (
                        i,
                        jnp.maximum(jnp.minimum(lob[i] + k, hib[i] - 1), 0),
                        0,
                    ),
                )
            ],
            out_specs=pl.BlockSpec(
                (1, 1, D), lambda i, k, lob, hib, effs: (i, 0, 0)
            ),
        ),
        out_shape=jax.ShapeDtypeStruct((B, 1, D), jnp.float32),
        compiler_params=pltpu.CompilerParams(
            dimension_semantics=("arbitrary", "arbitrary")
        ),
    )(sblk, hib, eff, features)
    tc_sums = tc_sums.reshape(B, D)

    out = pl.pallas_call(
        _combine_body,
        out_shape=jax.ShapeDtypeStruct((B, D), jnp.float32),
    )(sc_sums, tc_sums, inv[:, None])
    return out


# speedup vs baseline: 1.0524x; 1.0524x over previous
import jax
import jax.numpy as jnp
from jax import lax
from jax.experimental import pallas as pl
from jax.experimental.pallas import tpu as pltpu
from jax.experimental.pallas import tpu_sc as plsc

B = 16
L = 4096
D = 1024
NC = 2
NS = 16
DH = D // NC
CHUNK = 32
NLANE = 16
NCG = DH // NLANE

TROWS = 512
KMAX = L // TROWS


def _sc_body(features, s_hbm, out, s_v, buf, part, red, shared, final, sems):
    c = lax.axis_index("c")
    s = lax.axis_index("s")
    col0 = c * DH

    pltpu.sync_copy(s_hbm, s_v)

    iota = lax.iota(jnp.int32, NLANE)
    s_vec = s_v[...]
    zero = jnp.zeros((NLANE,), jnp.float32)

    def zero_row(i, _):
        def zero_cg(g, _):
            part[i, pl.ds(g * NLANE, NLANE)] = zero
            return 0
        return lax.fori_loop(0, NCG, zero_cg, 0)

    lax.fori_loop(0, B, zero_row, 0)

    nblk_v = s_vec // 8
    r0_v = 8 * ((s * nblk_v) // NS)
    rb1_v = 8 * (((s + 1) * nblk_v) // NS)
    nch_v = (rb1_v - r0_v + CHUNK - 1) // CHUNK
    total = jnp.sum(nch_v)

    def geti(v, i):
        return jnp.sum(jnp.where(iota == i, v, 0))

    def advance(i, k):
        def cond(st):
            i_, k_ = st
            return (i_ < B) & (k_ >= geti(nch_v, i_))

        def step(st):
            i_, _ = st
            return (i_ + 1, 0)

        return lax.while_loop(cond, step, (i, k))

    def chunk_start(i, k):
        r0 = geti(r0_v, i)
        rb1 = geti(rb1_v, i)
        islast = k == geti(nch_v, i) - 1
        start = jnp.where(
            islast, jnp.maximum(rb1 - CHUNK, 0), r0 + k * CHUNK
        )
        return pl.multiple_of(start, 8)

    def issue(i, k, par):
        return pltpu.async_copy(
            features.at[i, pl.ds(chunk_start(i, k), CHUNK), pl.ds(col0, DH)],
            buf.at[par],
            sems.at[par],
        )

    c0 = advance(0, 0)
    c1 = advance(c0[0], c0[1] + 1)
    c2 = advance(c1[0], c1[1] + 1)
    c3 = advance(c2[0], c2[1] + 1)
    for depth, cc in enumerate((c0, c1, c2)):
        @pl.when(total > depth)
        def _(cc=cc, depth=depth):
            issue(cc[0], cc[1], depth)

    def chunk_step(m, st):
        i, k, n1, n2, n3 = st
        par = m & 3

        @pl.when(m + 3 < total)
        def _():
            issue(n3[0], n3[1], (m + 3) & 3)

        pltpu.make_async_copy(
            features.at[i, pl.ds(chunk_start(i, k), CHUNK), pl.ds(col0, DH)],
            buf.at[par],
            sems.at[par],
        ).wait()

        islast = k == geti(nch_v, i) - 1

        @pl.when(jnp.logical_not(islast))
        def _():
            def per_cg(g, _):
                base = g * NLANE
                acc = part[i, pl.ds(base, NLANE)]
                for j in range(CHUNK):
                    acc = acc + buf[par, j, pl.ds(base, NLANE)]
                part[i, pl.ds(base, NLANE)] = acc
                return 0

            lax.fori_loop(0, NCG, per_cg, 0)

        @pl.when(islast)
        def _():
            start = chunk_start(i, k)
            lo = geti(r0_v, i) + k * CHUNK
            hi = geti(rb1_v, i)
            scales = [
                jnp.where((start + j >= lo) & (start + j < hi), 1.0, 0.0)
                for j in range(CHUNK)
            ]

            def per_cg(g, _):
                base = g * NLANE
                acc = part[i, pl.ds(base, NLANE)]
                for j in range(CHUNK):
                    acc = acc + buf[par, j, pl.ds(base, NLANE)] * scales[j]
                part[i, pl.ds(base, NLANE)] = acc
                return 0

            lax.fori_loop(0, NCG, per_cg, 0)

        return (n1[0], n1[1], n2, n3, advance(n3[0], n3[1] + 1))

    lax.fori_loop(0, total, chunk_step, (c0[0], c0[1], c1, c2, c3))

    pltpu.sync_copy(part, shared.at[s])
    plsc.subcore_barrier()

    pltpu.sync_copy(shared.at[:, s], part)

    def red_cg(g, _):
        base = g * NLANE
        acc = part[0, pl.ds(base, NLANE)]
        for w in range(1, NS):
            acc = acc + part[w, pl.ds(base, NLANE)]
        red[pl.ds(base, NLANE)] = acc
        return 0

    lax.fori_loop(0, NCG, red_cg, 0)

    pltpu.sync_copy(red, final.at[s])
    plsc.subcore_barrier()

    @pl.when(s == 0)
    def _():
        pltpu.sync_copy(final, out.at[:, pl.ds(col0, DH)])


def _tc_body(lob, hib, effs, feats_ref, out_ref):
    k = pl.program_id(1)
    i = pl.program_id(0)

    @pl.when(k == 0)
    def _():
        out_ref[...] = jnp.zeros(out_ref.shape, out_ref.dtype)

    lo = lob[i]
    hi = hib[i]

    @pl.when(lo + k < hi)
    def _():
        bidx = lo + k
        rows = bidx * TROWS + lax.broadcasted_iota(jnp.int32, (1, TROWS), 1)
        mask = (rows < effs[i]).astype(jnp.float32)
        out_ref[...] += jnp.dot(
            mask, feats_ref[0], preferred_element_type=jnp.float32
        )[None]


def _combine_body(a_ref, b_ref, inv_ref, o_ref):
    o_ref[...] = (a_ref[...] + b_ref[...]) * inv_ref[...]


@jax.jit
def kernel(features, lengths):
    eff = jnp.minimum(jnp.where(lengths <= 0, L, lengths), L).astype(jnp.int32)
    inv = 1.0 / eff.astype(jnp.float32)
    sblk = jnp.minimum((13 * eff + 10 * TROWS) // (20 * TROWS), eff // TROWS)
    s_rows = (TROWS // 8 * sblk) * 8

    mesh = plsc.VectorSubcoreMesh(core_axis_name="c", subcore_axis_name="s")
    sc_run = pl.kernel(
        _sc_body,
        out_type=jax.ShapeDtypeStruct((B, D), jnp.float32),
        mesh=mesh,
        scratch_types=[
            pltpu.VMEM((B,), jnp.int32),
            pltpu.VMEM((4, CHUNK, DH), jnp.float32),
            pltpu.VMEM((B, DH), jnp.float32),
            pltpu.VMEM((DH,), jnp.float32),
            pltpu.VMEM_SHARED((NS, B, DH), jnp.float32),
            pltpu.VMEM_SHARED((B, DH), jnp.float32),
            pltpu.SemaphoreType.DMA((4,)),
        ],
        compiler_params=pltpu.CompilerParams(needs_layout_passes=False),
    )
    sc_sums = sc_run(features, s_rows)

    hib = (eff + TROWS - 1) // TROWS
    tc_sums = pl.pallas_call(
        _tc_body,
        grid_spec=pltpu.PrefetchScalarGridSpec(
            num_scalar_prefetch=3,
            grid=(B, KMAX),
            in_specs=[
                pl.BlockSpec(
                    (1, TROWS, D),
                    lambda i, k, lob, hib, effs: (
                        i,
                        jnp.maximum(jnp.minimum(lob[i] + k, hib[i] - 1), 0),
                        0,
                    ),
                )
            ],
            out_specs=pl.BlockSpec(
                (1, 1, D), lambda i, k, lob, hib, effs: (i, 0, 0)
            ),
        ),
        out_shape=jax.ShapeDtypeStruct((B, 1, D), jnp.float32),
        compiler_params=pltpu.CompilerParams(
            dimension_semantics=("arbitrary", "arbitrary")
        ),
    )(sblk, hib, eff, features)
    tc_sums = tc_sums.reshape(B, D)

    out = pl.pallas_call(
        _combine_body,
        out_shape=jax.ShapeDtypeStruct((B, D), jnp.float32),
    )(sc_sums, tc_sums, inv[:, None])
    return out


# speedup vs baseline: 1.0706x; 1.0173x over previous
import jax
import jax.numpy as jnp
from jax import lax
from jax.experimental import pallas as pl
from jax.experimental.pallas import tpu as pltpu
from jax.experimental.pallas import tpu_sc as plsc

B = 16
L = 4096
D = 1024
NC = 2
NS = 16
DH = D // NC
CHUNK = 32
NLANE = 16
NCG = DH // NLANE

TROWS = 512
KMAX = L // TROWS


def _sc_body(features, s_hbm, out, s_v, buf, part, red, shared, final, sems):
    c = lax.axis_index("c")
    s = lax.axis_index("s")
    col0 = c * DH

    pltpu.sync_copy(s_hbm, s_v)

    iota = lax.iota(jnp.int32, NLANE)
    s_vec = s_v[...]
    zero = jnp.zeros((NLANE,), jnp.float32)

    def zero_row(i, _):
        def zero_cg(g, _):
            part[i, pl.ds(g * NLANE, NLANE)] = zero
            return 0
        return lax.fori_loop(0, NCG, zero_cg, 0)

    lax.fori_loop(0, B, zero_row, 0)

    nblk_v = s_vec // 8
    r0_v = 8 * ((s * nblk_v) // NS)
    rb1_v = 8 * (((s + 1) * nblk_v) // NS)
    nch_v = (rb1_v - r0_v + CHUNK - 1) // CHUNK
    total = jnp.sum(nch_v)

    def geti(v, i):
        return jnp.sum(jnp.where(iota == i, v, 0))

    def advance(i, k):
        def cond(st):
            i_, k_ = st
            return (i_ < B) & (k_ >= geti(nch_v, i_))

        def step(st):
            i_, _ = st
            return (i_ + 1, 0)

        return lax.while_loop(cond, step, (i, k))

    def chunk_start(i, k):
        r0 = geti(r0_v, i)
        rb1 = geti(rb1_v, i)
        islast = k == geti(nch_v, i) - 1
        start = jnp.where(
            islast, jnp.maximum(rb1 - CHUNK, 0), r0 + k * CHUNK
        )
        return pl.multiple_of(start, 8)

    def issue(i, k, par):
        return pltpu.async_copy(
            features.at[i, pl.ds(chunk_start(i, k), CHUNK), pl.ds(col0, DH)],
            buf.at[par],
            sems.at[par],
        )

    c0 = advance(0, 0)
    c1 = advance(c0[0], c0[1] + 1)
    c2 = advance(c1[0], c1[1] + 1)
    c3 = advance(c2[0], c2[1] + 1)
    for depth, cc in enumerate((c0, c1, c2)):
        @pl.when(total > depth)
        def _(cc=cc, depth=depth):
            issue(cc[0], cc[1], depth)

    def chunk_step(m, st):
        i, k, n1, n2, n3 = st
        par = m & 3

        @pl.when(m + 3 < total)
        def _():
            issue(n3[0], n3[1], (m + 3) & 3)

        pltpu.make_async_copy(
            features.at[i, pl.ds(chunk_start(i, k), CHUNK), pl.ds(col0, DH)],
            buf.at[par],
            sems.at[par],
        ).wait()

        islast = k == geti(nch_v, i) - 1

        @pl.when(jnp.logical_not(islast))
        def _():
            def per_cg(g, _):
                base = g * NLANE
                acc = part[i, pl.ds(base, NLANE)]
                for j in range(CHUNK):
                    acc = acc + buf[par, j, pl.ds(base, NLANE)]
                part[i, pl.ds(base, NLANE)] = acc
                return 0

            lax.fori_loop(0, NCG, per_cg, 0)

        @pl.when(islast)
        def _():
            start = chunk_start(i, k)
            lo = geti(r0_v, i) + k * CHUNK
            hi = geti(rb1_v, i)
            scales = [
                jnp.where((start + j >= lo) & (start + j < hi), 1.0, 0.0)
                for j in range(CHUNK)
            ]

            def per_cg(g, _):
                base = g * NLANE
                acc = part[i, pl.ds(base, NLANE)]
                for j in range(CHUNK):
                    acc = acc + buf[par, j, pl.ds(base, NLANE)] * scales[j]
                part[i, pl.ds(base, NLANE)] = acc
                return 0

            lax.fori_loop(0, NCG, per_cg, 0)

        return (n1[0], n1[1], n2, n3, advance(n3[0], n3[1] + 1))

    lax.fori_loop(0, total, chunk_step, (c0[0], c0[1], c1, c2, c3))

    pltpu.sync_copy(part, shared.at[s])
    plsc.subcore_barrier()

    pltpu.sync_copy(shared.at[:, s], part)

    def red_cg(g, _):
        base = g * NLANE
        acc = part[0, pl.ds(base, NLANE)]
        for w in range(1, NS):
            acc = acc + part[w, pl.ds(base, NLANE)]
        red[pl.ds(base, NLANE)] = acc
        return 0

    lax.fori_loop(0, NCG, red_cg, 0)

    pltpu.sync_copy(red, final.at[s])
    plsc.subcore_barrier()

    @pl.when(s == 0)
    def _():
        pltpu.sync_copy(final, out.at[:, pl.ds(col0, DH)])


def _tc_body(lob, hib, effs, feats_ref, out_ref):
    k = pl.program_id(1)
    i = pl.program_id(0)

    @pl.when(k == 0)
    def _():
        out_ref[...] = jnp.zeros(out_ref.shape, out_ref.dtype)

    lo = lob[i]
    hi = hib[i]

    @pl.when(lo + k < hi)
    def _():
        bidx = lo + k
        rows = bidx * TROWS + lax.broadcasted_iota(jnp.int32, (1, TROWS), 1)
        mask = (rows < effs[i]).astype(jnp.float32)
        out_ref[...] += jnp.dot(
            mask, feats_ref[0], preferred_element_type=jnp.float32
        )[None]


def _combine_body(a_ref, b_ref, inv_ref, o_ref):
    o_ref[...] = (a_ref[...] + b_ref[...]) * inv_ref[...]


@jax.jit
def kernel(features, lengths):
    eff = jnp.minimum(jnp.where(lengths <= 0, L, lengths), L).astype(jnp.int32)
    inv = 1.0 / eff.astype(jnp.float32)
    sblk = jnp.minimum((6 * eff + 5 * TROWS) // (10 * TROWS), eff // TROWS)
    s_rows = (TROWS // 8 * sblk) * 8

    mesh = plsc.VectorSubcoreMesh(core_axis_name="c", subcore_axis_name="s")
    sc_run = pl.kernel(
        _sc_body,
        out_type=jax.ShapeDtypeStruct((B, D), jnp.float32),
        mesh=mesh,
        scratch_types=[
            pltpu.VMEM((B,), jnp.int32),
            pltpu.VMEM((4, CHUNK, DH), jnp.float32),
            pltpu.VMEM((B, DH), jnp.float32),
            pltpu.VMEM((DH,), jnp.float32),
            pltpu.VMEM_SHARED((NS, B, DH), jnp.float32),
            pltpu.VMEM_SHARED((B, DH), jnp.float32),
            pltpu.SemaphoreType.DMA((4,)),
        ],
        compiler_params=pltpu.CompilerParams(needs_layout_passes=False),
    )
    sc_sums = sc_run(features, s_rows)

    hib = (eff + TROWS - 1) // TROWS
    tc_sums = pl.pallas_call(
        _tc_body,
        grid_spec=pltpu.PrefetchScalarGridSpec(
            num_scalar_prefetch=3,
            grid=(B, KMAX),
            in_specs=[
                pl.BlockSpec(
                    (1, TROWS, D),
                    lambda i, k, lob, hib, effs: (
                        i,
                        jnp.maximum(jnp.minimum(lob[i] + k, hib[i] - 1), 0),
                        0,
                    ),
                )
            ],
            out_specs=pl.BlockSpec(
                (1, 1, D), lambda i, k, lob, hib, effs: (i, 0, 0)
            ),
        ),
        out_shape=jax.ShapeDtypeStruct((B, 1, D), jnp.float32),
        compiler_params=pltpu.CompilerParams(
            dimension_semantics=("arbitrary", "arbitrary")
        ),
    )(sblk, hib, eff, features)
    tc_sums = tc_sums.reshape(B, D)

    out = pl.pallas_call(
        _combine_body,
        out_shape=jax.ShapeDtypeStruct((B, D), jnp.float32),
    )(sc_sums, tc_sums, inv[:, None])
    return out
